# baseline (speedup 1.0000x reference)
import functools

import jax
import jax.numpy as jnp
from jax import lax
from jax.experimental import pallas as pl
from jax.experimental.pallas import tpu as pltpu
from jax.experimental.pallas import tpu_sc as plsc

V = 1000
D = 64
DP = D // 2
J = 20
B = 4096
NC, NS = 2, 16
NW = NC * NS
BQ = B // NW
L = 16

_mesh = plsc.VectorSubcoreMesh(core_axis_name="c", subcore_axis_name="s")


@functools.partial(
    pl.kernel,
    mesh=_mesh,
    out_type=jax.ShapeDtypeStruct((J, D, B), jnp.float32),
    compiler_params=pltpu.CompilerParams(needs_layout_passes=False),
    scratch_types=[
        pltpu.VMEM((J, BQ), jnp.int32),
        pltpu.VMEM((V * DP,), jnp.int32),
        pltpu.VMEM((D, BQ), jnp.float32),
        pltpu.VMEM((D, BQ), jnp.float32),
        pltpu.SemaphoreType.DMA,
        pltpu.SemaphoreType.DMA,
    ],
)
def _emb_lookup(tok_hbm, tab_hbm, out_hbm, tok_v, tab_v, buf0, buf1, o0, o1):
    wid = lax.axis_index("s") * NC + lax.axis_index("c")
    b0 = wid * BQ
    bufs = (buf0, buf1)
    osems = (o0, o1)

    pltpu.sync_copy(tok_hbm.at[:, wid], tok_v)
    pltpu.sync_copy(tab_hbm, tab_v)

    def wait_write(p):
        pltpu.make_async_copy(
            bufs[p], out_hbm.at[0, :, pl.ds(b0, BQ)], osems[p]
        ).wait()

    def body(jj, _):
        for p in range(2):
            j = 2 * jj + p

            @pl.when(jj > 0)
            def _():
                wait_write(p)

            buf = bufs[p]

            @plsc.parallel_loop(0, BQ // L, unroll=8)
            def fi(i):
                idx = tok_v[j, pl.ds(i * L, L)]
                for k in range(DP):
                    w = plsc.load_gather(tab_v, [idx + k * V])
                    wb = plsc.bitcast(w, jnp.bfloat16)
                    a, b = plsc.unpack(wb, format=plsc.PackFormat.INTERLEAVED)
                    buf[2 * k, pl.ds(i * L, L)] = a
                    buf[2 * k + 1, pl.ds(i * L, L)] = b

            pltpu.async_copy(buf, out_hbm.at[j, :, pl.ds(b0, BQ)], osems[p])
        return 0

    lax.fori_loop(0, J // 2, body, 0)
    wait_write(0)
    wait_write(1)


def kernel(token_ids, embedding):
    tok = token_ids.astype(jnp.int32).T.reshape(J, NW, BQ)
    pairs = embedding.astype(jnp.bfloat16).reshape(V, DP, 2)
    tab = lax.bitcast_convert_type(pairs, jnp.int32).T.reshape(-1)
    out = _emb_lookup(tok, tab)
    return out.transpose(2, 0, 1)


# speedup vs baseline: 1.3338x; 1.3338x over previous
import functools

import jax
import jax.numpy as jnp
from jax import lax
from jax.experimental import pallas as pl
from jax.experimental.pallas import tpu as pltpu
from jax.experimental.pallas import tpu_sc as plsc

V = 1000
D = 64
DP = D // 2
J = 20
B = 4096
NC, NS = 2, 16
NW = NC * NS
BQ = B // NW
L = 16

_mesh = plsc.VectorSubcoreMesh(core_axis_name="c", subcore_axis_name="s")


@functools.partial(
    pl.kernel,
    mesh=_mesh,
    out_type=jax.ShapeDtypeStruct((J, D, B), jnp.float32),
    compiler_params=pltpu.CompilerParams(needs_layout_passes=False),
    scratch_types=[
        pltpu.VMEM((J, BQ), jnp.int32),
        pltpu.VMEM((V * DP,), jnp.int32),
        pltpu.VMEM((D, BQ), jnp.float32),
        pltpu.VMEM((D, BQ), jnp.float32),
        pltpu.SemaphoreType.DMA,
        pltpu.SemaphoreType.DMA,
    ],
)
def _emb_lookup(tok_hbm, tab_hbm, out_hbm, tok_v, tab_v, buf0, buf1, o0, o1):
    wid = lax.axis_index("s") * NC + lax.axis_index("c")
    b0 = wid * BQ
    bufs = (buf0, buf1)
    osems = (o0, o1)

    pltpu.sync_copy(tok_hbm.at[:, wid], tok_v)
    pltpu.sync_copy(tab_hbm, tab_v)

    def wait_write(p):
        pltpu.make_async_copy(
            bufs[p], out_hbm.at[0, :, pl.ds(b0, BQ)], osems[p]
        ).wait()

    def body(jj, _):
        for p in range(2):
            j = 2 * jj + p

            @pl.when(jj > 0)
            def _():
                wait_write(p)

            buf = bufs[p]

            @plsc.parallel_loop(0, BQ // L, unroll=2)
            def fi(i):
                idx = tok_v[j, pl.ds(i * L, L)]
                for k in range(DP):
                    w = plsc.load_gather(tab_v, [idx + k * V])
                    wb = plsc.bitcast(w, jnp.bfloat16)
                    a, b = plsc.unpack(wb, format=plsc.PackFormat.INTERLEAVED)
                    buf[2 * k, pl.ds(i * L, L)] = a
                    buf[2 * k + 1, pl.ds(i * L, L)] = b

            pltpu.async_copy(buf, out_hbm.at[j, :, pl.ds(b0, BQ)], osems[p])
        return 0

    lax.fori_loop(0, J // 2, body, 0)
    wait_write(0)
    wait_write(1)


def kernel(token_ids, embedding):
    tok = token_ids.astype(jnp.int32).T.reshape(J, NW, BQ)
    pairs = embedding.astype(jnp.bfloat16).reshape(V, DP, 2)
    tab = lax.bitcast_convert_type(pairs, jnp.int32).T.reshape(-1)
    out = _emb_lookup(tok, tab)
    return out.transpose(2, 0, 1)


# speedup vs baseline: 1.4041x; 1.0527x over previous
import functools

import jax
import jax.numpy as jnp
from jax import lax
from jax.experimental import pallas as pl
from jax.experimental.pallas import tpu as pltpu
from jax.experimental.pallas import tpu_sc as plsc

V = 1000
D = 64
DP = D // 2
J = 20
B = 4096
NC, NS = 2, 16
NW = NC * NS
BQ = B // NW
L = 16

_mesh = plsc.VectorSubcoreMesh(core_axis_name="c", subcore_axis_name="s")


@functools.partial(
    pl.kernel,
    mesh=_mesh,
    out_type=jax.ShapeDtypeStruct((J, D, B), jnp.float32),
    compiler_params=pltpu.CompilerParams(needs_layout_passes=False),
    scratch_types=[
        pltpu.VMEM((J, BQ), jnp.int32),
        pltpu.VMEM((V * DP,), jnp.int32),
        pltpu.VMEM((D, BQ), jnp.float32),
        pltpu.VMEM((D, BQ), jnp.float32),
        pltpu.SemaphoreType.DMA,
        pltpu.SemaphoreType.DMA,
    ],
)
def _emb_lookup(tok_hbm, tab_hbm, out_hbm, tok_v, tab_v, buf0, buf1, o0, o1):
    wid = lax.axis_index("s") * NC + lax.axis_index("c")
    b0 = wid * BQ
    bufs = (buf0, buf1)
    osems = (o0, o1)

    pltpu.sync_copy(tok_hbm.at[:, wid], tok_v)
    pltpu.sync_copy(tab_hbm, tab_v)

    def wait_write(p):
        pltpu.make_async_copy(
            bufs[p], out_hbm.at[0, :, pl.ds(b0, BQ)], osems[p]
        ).wait()

    def body(jj, _):
        for p in range(2):
            j = 2 * jj + p

            @pl.when(jj > 0)
            def _():
                wait_write(p)

            buf = bufs[p]

            @plsc.parallel_loop(0, BQ // L, unroll=4)
            def fi(i):
                idx = tok_v[j, pl.ds(i * L, L)]
                for k in range(DP):
                    w = plsc.load_gather(tab_v, [idx + k * V])
                    wb = plsc.bitcast(w, jnp.bfloat16)
                    a, b = plsc.unpack(wb, format=plsc.PackFormat.INTERLEAVED)
                    buf[2 * k, pl.ds(i * L, L)] = a
                    buf[2 * k + 1, pl.ds(i * L, L)] = b

            pltpu.async_copy(buf, out_hbm.at[j, :, pl.ds(b0, BQ)], osems[p])
        return 0

    lax.fori_loop(0, J // 2, body, 0)
    wait_write(0)
    wait_write(1)


def kernel(token_ids, embedding):
    tok = token_ids.astype(jnp.int32).T.reshape(J, NW, BQ)
    pairs = embedding.astype(jnp.bfloat16).reshape(V, DP, 2)
    tab = lax.bitcast_convert_type(pairs, jnp.int32).T.reshape(-1)
    out = _emb_lookup(tok, tab)
    return out.transpose(2, 0, 1)


# speedup vs baseline: 1.4304x; 1.0188x over previous
import functools

import jax
import jax.numpy as jnp
from jax import lax
from jax.experimental import pallas as pl
from jax.experimental.pallas import tpu as pltpu
from jax.experimental.pallas import tpu_sc as plsc

V = 1000
D = 64
DP = D // 2
J = 20
B = 4096
NC, NS = 2, 16
NW = NC * NS
BQ = B // NW
L = 16

_mesh = plsc.VectorSubcoreMesh(core_axis_name="c", subcore_axis_name="s")


@functools.partial(
    pl.kernel,
    mesh=_mesh,
    out_type=jax.ShapeDtypeStruct((J, D, B), jnp.float32),
    compiler_params=pltpu.CompilerParams(needs_layout_passes=False),
    scratch_types=[
        pltpu.VMEM((J, BQ), jnp.int32),
        pltpu.VMEM((V * DP,), jnp.int32),
        pltpu.VMEM((D, BQ), jnp.float32),
        pltpu.VMEM((D, BQ), jnp.float32),
        pltpu.SemaphoreType.DMA,
        pltpu.SemaphoreType.DMA,
    ],
)
def _emb_lookup(tok_hbm, tab_hbm, out_hbm, tok_v, tab_v, buf0, buf1, o0, o1):
    wid = lax.axis_index("s") * NC + lax.axis_index("c")
    b0 = wid * BQ
    bufs = (buf0, buf1)
    osems = (o0, o1)

    tok_cp = pltpu.async_copy(tok_hbm.at[:, wid], tok_v, o0)
    tab_cp = pltpu.async_copy(tab_hbm, tab_v, o1)
    tok_cp.wait()
    tab_cp.wait()

    def wait_write(p):
        pltpu.make_async_copy(
            bufs[p], out_hbm.at[0, :, pl.ds(b0, BQ)], osems[p]
        ).wait()

    def body(jj, _):
        for p in range(2):
            j = 2 * jj + p

            @pl.when(jj > 0)
            def _():
                wait_write(p)

            buf = bufs[p]

            @plsc.parallel_loop(0, BQ // L, unroll=4)
            def fi(i):
                idx = tok_v[j, pl.ds(i * L, L)]
                for k in range(DP):
                    w = plsc.load_gather(tab_v, [idx + k * V])
                    wb = plsc.bitcast(w, jnp.bfloat16)
                    a, b = plsc.unpack(wb, format=plsc.PackFormat.INTERLEAVED)
                    buf[2 * k, pl.ds(i * L, L)] = a
                    buf[2 * k + 1, pl.ds(i * L, L)] = b

            pltpu.async_copy(buf, out_hbm.at[j, :, pl.ds(b0, BQ)], osems[p])
        return 0

    lax.fori_loop(0, J // 2, body, 0)
    wait_write(0)
    wait_write(1)


def kernel(token_ids, embedding):
    tok = token_ids.astype(jnp.int32).T.reshape(J, NW, BQ)
    pairs = embedding.astype(jnp.bfloat16).reshape(V, DP, 2)
    tab = lax.bitcast_convert_type(pairs, jnp.int32).T.reshape(-1)
    out = _emb_lookup(tok, tab)
    return out.transpose(2, 0, 1)
